# baseline (speedup 1.0000x reference)
import functools

import jax
import jax.numpy as jnp
from jax import lax
from jax.experimental import pallas as pl
from jax.experimental.pallas import tpu as pltpu
from jax.experimental.pallas import tpu_sc as plsc

B = 4096
H = 200
E = 64
OUTD = 128
HALF = H // 2
VOCAB_HALF = 500000


def _make_sc_pool():
    info = plsc.get_sparse_core_info()
    nc, ns = info.num_cores, info.num_subcores
    nw = nc * ns
    bpw = B // nw
    mesh = plsc.VectorSubcoreMesh(core_axis_name="c", subcore_axis_name="s")

    @functools.partial(
        pl.kernel,
        out_type=jax.ShapeDtypeStruct((B // 2, 2 * E), jnp.float32),
        mesh=mesh,
        scratch_types=[
            pltpu.VMEM((bpw, 2, HALF), jnp.int32),
            pltpu.VMEM((2, 2, HALF), jnp.int32),
            pltpu.VMEM((2, 2, HALF, 2 * E), jnp.float32),
            pltpu.VMEM((bpw // 2, 2 * E), jnp.float32),
            pltpu.SemaphoreType.DMA,
            pltpu.SemaphoreType.DMA,
        ],
    )
    def pool(x_hbm, table_hbm, out_hbm,
             idx_v, pidx_v, rows_v, pooled_v, sem0, sem1):
        sems = (sem0, sem1)
        wid = lax.axis_index("s") * nc + lax.axis_index("c")
        base = wid * bpw
        pltpu.sync_copy(x_hbm.at[pl.ds(base, bpw)], idx_v)

        group_starts = list(range(0, HALF - 16, 16)) + [HALF - 16]

        def start(b, par):
            for h in range(2):
                for r0 in group_starts:
                    pidx_v[par, h, pl.ds(r0, 16)] = (
                        idx_v[b, h, pl.ds(r0, 16)] >> 1
                    )
                pltpu.async_copy(
                    table_hbm.at[pidx_v.at[par, h]], rows_v.at[par, h],
                    sems[par],
                )

        def wait(b, par):
            for h in range(2):
                pltpu.make_async_copy(
                    table_hbm.at[pidx_v.at[par, h]], rows_v.at[par, h],
                    sems[par],
                ).wait()

        start(0, 0)

        def outer(g, _):
            for par in range(2):
                b = 2 * g + par

                @pl.when(b + 1 < bpw)
                def _():
                    start(b + 1, (par + 1) % 2)

                wait(b, par)

                def add_rows(accs, h, r0, ks):
                    offv = (idx_v[b, h, pl.ds(r0, 16)] & 1) << 6
                    accs = list(accs)
                    for k in ks:
                        off = offv[k]
                        for j in range(E // 16):
                            i = (h * (E // 16) + j) * 2 + (k % 2)
                            accs[i] = accs[i] + rows_v[
                                par, h, r0 + k, pl.ds(off + j * 16, 16)
                            ]
                    return tuple(accs)

                def group(g16, accs):
                    for h in range(2):
                        accs = add_rows(accs, h, g16 * 16, range(16))
                    return accs

                zero = jnp.zeros((16,), jnp.float32)
                accs = lax.fori_loop(
                    0, (HALF // 16), group, (zero,) * (4 * (E // 16))
                )
                for h in range(2):
                    accs = add_rows(accs, h, HALF - 16, range(12, 16))
                for j in range(E // 16):
                    s = (accs[j * 2] + accs[j * 2 + 1]) + (
                        accs[(E // 16 + j) * 2] + accs[(E // 16 + j) * 2 + 1]
                    )
                    pooled_v[g, pl.ds(par * E + j * 16, 16)] = s
            return 0

        lax.fori_loop(0, bpw // 2, outer, 0)
        pltpu.sync_copy(pooled_v, out_hbm.at[pl.ds(wid * (bpw // 2), bpw // 2)])

    return pool


def _tc_proj(pooled_sum, W, b):
    blk = 512

    def body(p_ref, w_ref, b_ref, o_ref):
        o_ref[...] = (
            jnp.dot(
                p_ref[...] * (1.0 / H), w_ref[...],
                preferred_element_type=jnp.float32,
            )
            + b_ref[...]
        )

    return pl.pallas_call(
        body,
        grid=(B // blk,),
        in_specs=[
            pl.BlockSpec((blk, E), lambda i: (i, 0)),
            pl.BlockSpec((E, OUTD), lambda i: (0, 0)),
            pl.BlockSpec((1, OUTD), lambda i: (0, 0)),
        ],
        out_specs=pl.BlockSpec((blk, OUTD), lambda i: (i, 0)),
        out_shape=jax.ShapeDtypeStruct((B, OUTD), jnp.float32),
    )(pooled_sum, W, b.reshape(1, OUTD))


def kernel(x, table, W, b):
    x3 = x.astype(jnp.int32).reshape(B, 2, HALF)
    table2 = table.reshape(VOCAB_HALF, 2 * E)
    pooled_sum = _make_sc_pool()(x3, table2).reshape(B, E)
    return _tc_proj(pooled_sum, W, b)
